# baseline (speedup 1.0000x reference)
import functools

import jax
import jax.numpy as jnp
from jax import lax
from jax.experimental import pallas as pl
from jax.experimental.pallas import tpu as pltpu
from jax.experimental.pallas import tpu_sc as plsc

VOCAB_P1 = 100001
EMB = 128
BATCH = 16384
WINDOW = 5
N_CLASS = 50

NC = 2
NS = 16
NW = NC * NS

B_PER_W = BATCH // NW
CHUNK = 128
NB = B_PER_W // CHUNK
K = 2
NBUF = 3
WSPLIT = 3


def _make_gather(nwin):
  nchunk = nwin * NB
  ngrp = nchunk // K

  def body(idx_hbm, table_hbm, out_hbm, idx_v, *bufs_and_sems):
    wid = lax.axis_index("s") * NC + lax.axis_index("c")
    base = wid * B_PER_W
    pltpu.sync_copy(idx_hbm.at[wid], idx_v)
    rows = bufs_and_sems[:NBUF * K]
    gsems = bufs_and_sems[NBUF * K:NBUF * K + NBUF]
    wsems = bufs_and_sems[NBUF * K + NBUF:]
    bufs = [(rows[g * K:(g + 1) * K], gsems[g], wsems[g]) for g in range(NBUF)]

    def fire_gathers(grp):
      rs, gs, _ = bufs[grp % NBUF]
      return [pltpu.async_copy(table_hbm.at[idx_v.at[grp * K + k]], rs[k], gs)
              for k in range(K)]

    def fire_writes(grp):
      rs, _, ws = bufs[grp % NBUF]
      hs = []
      for k in range(K):
        j = grp * K + k
        w, c = j // NB, j % NB
        hs.append(pltpu.async_copy(
            rs[k], out_hbm.at[w, pl.ds(base + c * CHUNK, CHUNK)], ws))
      return hs

    g_handles = {g: fire_gathers(g) for g in range(min(NBUF - 1, ngrp))}
    w_handles = {}
    for grp in range(ngrp):
      nxt = grp + NBUF - 1
      if nxt < ngrp:
        if grp >= 1:
          for h in w_handles[grp - 1]:
            h.wait()
        g_handles[nxt] = fire_gathers(nxt)
      for h in g_handles[grp]:
        h.wait()
      w_handles[grp] = fire_writes(grp)
    for grp in range(max(0, ngrp - NBUF), ngrp):
      if grp in w_handles:
        for h in w_handles[grp]:
          h.wait()

  return functools.partial(
      pl.kernel,
      out_type=jax.ShapeDtypeStruct((nwin, BATCH, EMB), jnp.float32),
      mesh=plsc.VectorSubcoreMesh(core_axis_name="c", subcore_axis_name="s"),
      scratch_types=(
          [pltpu.VMEM((nchunk, CHUNK), jnp.int32)]
          + [pltpu.VMEM((CHUNK, EMB), jnp.float32) for _ in range(NBUF * K)]
          + [pltpu.SemaphoreType.DMA for _ in range(2 * NBUF)]
      ),
  )(body)


_sc_gather_a = _make_gather(WSPLIT)
_sc_gather_b = _make_gather(WINDOW - WSPLIT)


BM = 2048


def _partial_dots(g_ref, w_ref, nwin, acc):
  for w in range(nwin):
    acc = acc + lax.dot_general(
        g_ref[w], w_ref[w],
        dimension_numbers=(((1,), (1,)), ((), ())),
        preferred_element_type=jnp.float32,
    )
  return acc


def _matmul_a_body(g_ref, w_ref, b_ref, out_ref):
  out_ref[...] = _partial_dots(g_ref, w_ref, WSPLIT, b_ref[...])


def _matmul_b_body(g_ref, w_ref, prev_ref, out_ref):
  out_ref[...] = _partial_dots(g_ref, w_ref, WINDOW - WSPLIT, prev_ref[...])


def _tc_matmul_a(g, wr, b2d):
  return pl.pallas_call(
      _matmul_a_body,
      grid=(BATCH // BM,),
      in_specs=[
          pl.BlockSpec((WSPLIT, BM, EMB), lambda i: (0, i, 0)),
          pl.BlockSpec((WSPLIT, N_CLASS, EMB), lambda i: (0, 0, 0)),
          pl.BlockSpec((1, N_CLASS), lambda i: (0, 0)),
      ],
      out_specs=pl.BlockSpec((BM, N_CLASS), lambda i: (i, 0)),
      out_shape=jax.ShapeDtypeStruct((BATCH, N_CLASS), jnp.float32),
  )(g, wr, b2d)


def _tc_matmul_b(g, wr, prev):
  nw = WINDOW - WSPLIT
  return pl.pallas_call(
      _matmul_b_body,
      grid=(BATCH // BM,),
      in_specs=[
          pl.BlockSpec((nw, BM, EMB), lambda i: (0, i, 0)),
          pl.BlockSpec((nw, N_CLASS, EMB), lambda i: (0, 0, 0)),
          pl.BlockSpec((BM, N_CLASS), lambda i: (i, 0)),
      ],
      out_specs=pl.BlockSpec((BM, N_CLASS), lambda i: (i, 0)),
      out_shape=jax.ShapeDtypeStruct((BATCH, N_CLASS), jnp.float32),
  )(g, wr, prev)


def _make_idx(xt, lo, n):
  return (xt[lo:lo + n]
          .reshape(n, NW, NB, CHUNK)
          .transpose(1, 0, 2, 3)
          .reshape(NW, n * NB, CHUNK))


@jax.jit
def kernel(x, table, W, b):
  xt = x.astype(jnp.int32).T
  wr = W.reshape(N_CLASS, WINDOW, EMB).transpose(1, 0, 2)
  g_a = _sc_gather_a(_make_idx(xt, 0, WSPLIT), table)
  g_b = _sc_gather_b(_make_idx(xt, WSPLIT, WINDOW - WSPLIT), table)
  part = _tc_matmul_a(g_a, wr[:WSPLIT], b.reshape(1, N_CLASS))
  return _tc_matmul_b(g_b, wr[WSPLIT:], part)


# speedup vs baseline: 1.0825x; 1.0825x over previous
import functools

import jax
import jax.numpy as jnp
from jax import lax
from jax.experimental import pallas as pl
from jax.experimental.pallas import tpu as pltpu
from jax.experimental.pallas import tpu_sc as plsc

VOCAB_P1 = 100001
EMB = 128
BATCH = 16384
WINDOW = 5
N_CLASS = 50

NC = 2
NS = 16
NW = NC * NS

B_PER_W = BATCH // NW
CHUNK = 128
NB = B_PER_W // CHUNK
K = 2
NBUF = 3
WSPLIT = 5


def _make_gather(nwin):
  nchunk = nwin * NB
  ngrp = nchunk // K

  def body(idx_hbm, table_hbm, out_hbm, idx_v, *bufs_and_sems):
    wid = lax.axis_index("s") * NC + lax.axis_index("c")
    base = wid * B_PER_W
    pltpu.sync_copy(idx_hbm.at[wid], idx_v)
    rows = bufs_and_sems[:NBUF * K]
    gsems = bufs_and_sems[NBUF * K:NBUF * K + NBUF]
    wsems = bufs_and_sems[NBUF * K + NBUF:]
    bufs = [(rows[g * K:(g + 1) * K], gsems[g], wsems[g]) for g in range(NBUF)]

    def fire_gathers(grp):
      rs, gs, _ = bufs[grp % NBUF]
      return [pltpu.async_copy(table_hbm.at[idx_v.at[grp * K + k]], rs[k], gs)
              for k in range(K)]

    def fire_writes(grp):
      rs, _, ws = bufs[grp % NBUF]
      hs = []
      for k in range(K):
        j = grp * K + k
        w, c = j // NB, j % NB
        hs.append(pltpu.async_copy(
            rs[k], out_hbm.at[w, pl.ds(base + c * CHUNK, CHUNK)], ws))
      return hs

    g_handles = {g: fire_gathers(g) for g in range(min(NBUF - 1, ngrp))}
    w_handles = {}
    for grp in range(ngrp):
      nxt = grp + NBUF - 1
      if nxt < ngrp:
        if grp >= 1:
          for h in w_handles[grp - 1]:
            h.wait()
        g_handles[nxt] = fire_gathers(nxt)
      for h in g_handles[grp]:
        h.wait()
      w_handles[grp] = fire_writes(grp)
    for grp in range(max(0, ngrp - NBUF), ngrp):
      if grp in w_handles:
        for h in w_handles[grp]:
          h.wait()

  return functools.partial(
      pl.kernel,
      out_type=jax.ShapeDtypeStruct((nwin, BATCH, EMB), jnp.float32),
      mesh=plsc.VectorSubcoreMesh(core_axis_name="c", subcore_axis_name="s"),
      scratch_types=(
          [pltpu.VMEM((nchunk, CHUNK), jnp.int32)]
          + [pltpu.VMEM((CHUNK, EMB), jnp.float32) for _ in range(NBUF * K)]
          + [pltpu.SemaphoreType.DMA for _ in range(2 * NBUF)]
      ),
  )(body)


_sc_gather_a = _make_gather(WSPLIT)
if WSPLIT < WINDOW:
  _sc_gather_b = _make_gather(WINDOW - WSPLIT)


BM = 2048


def _partial_dots(g_ref, w_ref, lo, nwin, acc):
  for w in range(nwin):
    acc = acc + lax.dot_general(
        g_ref[w], w_ref[:, pl.ds((lo + w) * EMB, EMB)],
        dimension_numbers=(((1,), (1,)), ((), ())),
        preferred_element_type=jnp.float32,
    )
  return acc


def _matmul_a_body(g_ref, w_ref, b_ref, out_ref):
  out_ref[...] = _partial_dots(g_ref, w_ref, 0, WSPLIT, b_ref[...])


def _matmul_b_body(g_ref, w_ref, prev_ref, out_ref):
  out_ref[...] = _partial_dots(g_ref, w_ref, WSPLIT, WINDOW - WSPLIT,
                               prev_ref[...])


def _tc_matmul_a(g, W, b2d):
  return pl.pallas_call(
      _matmul_a_body,
      grid=(BATCH // BM,),
      in_specs=[
          pl.BlockSpec((WSPLIT, BM, EMB), lambda i: (0, i, 0)),
          pl.BlockSpec((N_CLASS, WINDOW * EMB), lambda i: (0, 0)),
          pl.BlockSpec((1, N_CLASS), lambda i: (0, 0)),
      ],
      out_specs=pl.BlockSpec((BM, N_CLASS), lambda i: (i, 0)),
      out_shape=jax.ShapeDtypeStruct((BATCH, N_CLASS), jnp.float32),
  )(g, W, b2d)


def _tc_matmul_b(g, W, prev):
  nw = WINDOW - WSPLIT
  return pl.pallas_call(
      _matmul_b_body,
      grid=(BATCH // BM,),
      in_specs=[
          pl.BlockSpec((nw, BM, EMB), lambda i: (0, i, 0)),
          pl.BlockSpec((N_CLASS, WINDOW * EMB), lambda i: (0, 0)),
          pl.BlockSpec((BM, N_CLASS), lambda i: (i, 0)),
      ],
      out_specs=pl.BlockSpec((BM, N_CLASS), lambda i: (i, 0)),
      out_shape=jax.ShapeDtypeStruct((BATCH, N_CLASS), jnp.float32),
  )(g, W, prev)


def _make_idx(xt, lo, n):
  return (xt[lo:lo + n]
          .reshape(n, NW, NB, CHUNK)
          .transpose(1, 0, 2, 3)
          .reshape(NW, n * NB, CHUNK))


@jax.jit
def kernel(x, table, W, b):
  xt = x.astype(jnp.int32).T
  g_a = _sc_gather_a(_make_idx(xt, 0, WSPLIT), table)
  part = _tc_matmul_a(g_a, W, b.reshape(1, N_CLASS))
  if WSPLIT == WINDOW:
    return part
  g_b = _sc_gather_b(_make_idx(xt, WSPLIT, WINDOW - WSPLIT), table)
  return _tc_matmul_b(g_b, W, part)


# speedup vs baseline: 1.0966x; 1.0131x over previous
import functools

import jax
import jax.numpy as jnp
from jax import lax
from jax.experimental import pallas as pl
from jax.experimental.pallas import tpu as pltpu
from jax.experimental.pallas import tpu_sc as plsc

VOCAB_P1 = 100001
EMB = 128
BATCH = 16384
WINDOW = 5
N_CLASS = 50

NC = 2
NS = 16
NW = NC * NS

B_PER_W = BATCH // NW
CHUNK = 128
NB = B_PER_W // CHUNK
K = 2
NBUF = 3
WSPLIT = 5


def _make_gather(nwin):
  nchunk = nwin * NB
  ngrp = nchunk // K

  def body(idx_hbm, table_hbm, out_hbm, idx_v, *bufs_and_sems):
    wid = lax.axis_index("s") * NC + lax.axis_index("c")
    base = wid * B_PER_W
    pltpu.sync_copy(idx_hbm.at[wid], idx_v)
    rows = bufs_and_sems[:NBUF * K]
    gsems = bufs_and_sems[NBUF * K:NBUF * K + NBUF]
    wsems = bufs_and_sems[NBUF * K + NBUF:]
    bufs = [(rows[g * K:(g + 1) * K], gsems[g], wsems[g]) for g in range(NBUF)]

    def fire_gathers(grp):
      rs, gs, _ = bufs[grp % NBUF]
      return [pltpu.async_copy(table_hbm.at[idx_v.at[grp * K + k]], rs[k], gs)
              for k in range(K)]

    def fire_writes(grp):
      rs, _, ws = bufs[grp % NBUF]
      hs = []
      for k in range(K):
        j = grp * K + k
        w, c = j // NB, j % NB
        hs.append(pltpu.async_copy(
            rs[k], out_hbm.at[w, pl.ds(base + c * CHUNK, CHUNK)], ws))
      return hs

    g_handles = {g: fire_gathers(g) for g in range(min(NBUF - 1, ngrp))}
    w_handles = {}
    for grp in range(ngrp):
      nxt = grp + NBUF - 1
      if nxt < ngrp:
        if grp >= 1:
          for h in w_handles[grp - 1]:
            h.wait()
        g_handles[nxt] = fire_gathers(nxt)
      for h in g_handles[grp]:
        h.wait()
      w_handles[grp] = fire_writes(grp)
    for grp in range(max(0, ngrp - NBUF), ngrp):
      if grp in w_handles:
        for h in w_handles[grp]:
          h.wait()

  return functools.partial(
      pl.kernel,
      out_type=jax.ShapeDtypeStruct((nwin, BATCH, EMB), jnp.float32),
      mesh=plsc.VectorSubcoreMesh(core_axis_name="c", subcore_axis_name="s"),
      scratch_types=(
          [pltpu.VMEM((nchunk, CHUNK), jnp.int32)]
          + [pltpu.VMEM((CHUNK, EMB), jnp.float32) for _ in range(NBUF * K)]
          + [pltpu.SemaphoreType.DMA for _ in range(2 * NBUF)]
      ),
  )(body)


_sc_gather_a = _make_gather(WSPLIT)
if WSPLIT < WINDOW:
  _sc_gather_b = _make_gather(WINDOW - WSPLIT)


BM = 4096


def _partial_dots(g_ref, w_ref, lo, nwin, acc):
  for w in range(nwin):
    acc = acc + lax.dot_general(
        g_ref[w], w_ref[:, pl.ds((lo + w) * EMB, EMB)],
        dimension_numbers=(((1,), (1,)), ((), ())),
        preferred_element_type=jnp.float32,
    )
  return acc


def _matmul_a_body(g_ref, w_ref, b_ref, out_ref):
  out_ref[...] = _partial_dots(g_ref, w_ref, 0, WSPLIT, b_ref[...])


def _matmul_b_body(g_ref, w_ref, prev_ref, out_ref):
  out_ref[...] = _partial_dots(g_ref, w_ref, WSPLIT, WINDOW - WSPLIT,
                               prev_ref[...])


def _tc_matmul_a(g, W, b2d):
  return pl.pallas_call(
      _matmul_a_body,
      grid=(BATCH // BM,),
      in_specs=[
          pl.BlockSpec((WSPLIT, BM, EMB), lambda i: (0, i, 0)),
          pl.BlockSpec((N_CLASS, WINDOW * EMB), lambda i: (0, 0)),
          pl.BlockSpec((1, N_CLASS), lambda i: (0, 0)),
      ],
      out_specs=pl.BlockSpec((BM, N_CLASS), lambda i: (i, 0)),
      out_shape=jax.ShapeDtypeStruct((BATCH, N_CLASS), jnp.float32),
  )(g, W, b2d)


def _tc_matmul_b(g, W, prev):
  nw = WINDOW - WSPLIT
  return pl.pallas_call(
      _matmul_b_body,
      grid=(BATCH // BM,),
      in_specs=[
          pl.BlockSpec((nw, BM, EMB), lambda i: (0, i, 0)),
          pl.BlockSpec((N_CLASS, WINDOW * EMB), lambda i: (0, 0)),
          pl.BlockSpec((BM, N_CLASS), lambda i: (i, 0)),
      ],
      out_specs=pl.BlockSpec((BM, N_CLASS), lambda i: (i, 0)),
      out_shape=jax.ShapeDtypeStruct((BATCH, N_CLASS), jnp.float32),
  )(g, W, prev)


def _make_idx(xt, lo, n):
  return (xt[lo:lo + n]
          .reshape(n, NW, NB, CHUNK)
          .transpose(1, 0, 2, 3)
          .reshape(NW, n * NB, CHUNK))


@jax.jit
def kernel(x, table, W, b):
  xt = x.astype(jnp.int32).T
  g_a = _sc_gather_a(_make_idx(xt, 0, WSPLIT), table)
  part = _tc_matmul_a(g_a, W, b.reshape(1, N_CLASS))
  if WSPLIT == WINDOW:
    return part
  g_b = _sc_gather_b(_make_idx(xt, WSPLIT, WINDOW - WSPLIT), table)
  return _tc_matmul_b(g_b, W, part)


# speedup vs baseline: 1.1126x; 1.0145x over previous
import functools

import jax
import jax.numpy as jnp
from jax import lax
from jax.experimental import pallas as pl
from jax.experimental.pallas import tpu as pltpu
from jax.experimental.pallas import tpu_sc as plsc

VOCAB_P1 = 100001
EMB = 128
BATCH = 16384
WINDOW = 5
N_CLASS = 50

NC = 2
NS = 16
NW = NC * NS

B_PER_W = BATCH // NW
CHUNK = 128
NB = B_PER_W // CHUNK
K = 1
NBUF = 6
WSPLIT = 5


def _make_gather(nwin):
  nchunk = nwin * NB
  ngrp = nchunk // K

  def body(idx_hbm, table_hbm, out_hbm, idx_v, *bufs_and_sems):
    wid = lax.axis_index("s") * NC + lax.axis_index("c")
    base = wid * B_PER_W
    pltpu.sync_copy(idx_hbm.at[wid], idx_v)
    rows = bufs_and_sems[:NBUF * K]
    gsems = bufs_and_sems[NBUF * K:NBUF * K + NBUF]
    wsems = bufs_and_sems[NBUF * K + NBUF:]
    bufs = [(rows[g * K:(g + 1) * K], gsems[g], wsems[g]) for g in range(NBUF)]

    def fire_gathers(grp):
      rs, gs, _ = bufs[grp % NBUF]
      return [pltpu.async_copy(table_hbm.at[idx_v.at[grp * K + k]], rs[k], gs)
              for k in range(K)]

    def fire_writes(grp):
      rs, _, ws = bufs[grp % NBUF]
      hs = []
      for k in range(K):
        j = grp * K + k
        w, c = j // NB, j % NB
        hs.append(pltpu.async_copy(
            rs[k], out_hbm.at[w, pl.ds(base + c * CHUNK, CHUNK)], ws))
      return hs

    g_handles = {g: fire_gathers(g) for g in range(min(NBUF - 1, ngrp))}
    w_handles = {}
    for grp in range(ngrp):
      nxt = grp + NBUF - 1
      if nxt < ngrp:
        if grp >= 1:
          for h in w_handles[grp - 1]:
            h.wait()
        g_handles[nxt] = fire_gathers(nxt)
      for h in g_handles[grp]:
        h.wait()
      w_handles[grp] = fire_writes(grp)
    for grp in range(max(0, ngrp - NBUF), ngrp):
      if grp in w_handles:
        for h in w_handles[grp]:
          h.wait()

  return functools.partial(
      pl.kernel,
      out_type=jax.ShapeDtypeStruct((nwin, BATCH, EMB), jnp.float32),
      mesh=plsc.VectorSubcoreMesh(core_axis_name="c", subcore_axis_name="s"),
      scratch_types=(
          [pltpu.VMEM((nchunk, CHUNK), jnp.int32)]
          + [pltpu.VMEM((CHUNK, EMB), jnp.float32) for _ in range(NBUF * K)]
          + [pltpu.SemaphoreType.DMA for _ in range(2 * NBUF)]
      ),
  )(body)


_sc_gather_a = _make_gather(WSPLIT)
if WSPLIT < WINDOW:
  _sc_gather_b = _make_gather(WINDOW - WSPLIT)


BM = 4096


def _partial_dots(g_ref, w_ref, lo, nwin, acc):
  for w in range(nwin):
    acc = acc + lax.dot_general(
        g_ref[w], w_ref[:, pl.ds((lo + w) * EMB, EMB)],
        dimension_numbers=(((1,), (1,)), ((), ())),
        preferred_element_type=jnp.float32,
    )
  return acc


def _matmul_a_body(g_ref, w_ref, b_ref, out_ref):
  out_ref[...] = _partial_dots(g_ref, w_ref, 0, WSPLIT, b_ref[...])


def _matmul_b_body(g_ref, w_ref, prev_ref, out_ref):
  out_ref[...] = _partial_dots(g_ref, w_ref, WSPLIT, WINDOW - WSPLIT,
                               prev_ref[...])


def _tc_matmul_a(g, W, b2d):
  return pl.pallas_call(
      _matmul_a_body,
      grid=(BATCH // BM,),
      in_specs=[
          pl.BlockSpec((WSPLIT, BM, EMB), lambda i: (0, i, 0)),
          pl.BlockSpec((N_CLASS, WINDOW * EMB), lambda i: (0, 0)),
          pl.BlockSpec((1, N_CLASS), lambda i: (0, 0)),
      ],
      out_specs=pl.BlockSpec((BM, N_CLASS), lambda i: (i, 0)),
      out_shape=jax.ShapeDtypeStruct((BATCH, N_CLASS), jnp.float32),
  )(g, W, b2d)


def _tc_matmul_b(g, W, prev):
  nw = WINDOW - WSPLIT
  return pl.pallas_call(
      _matmul_b_body,
      grid=(BATCH // BM,),
      in_specs=[
          pl.BlockSpec((nw, BM, EMB), lambda i: (0, i, 0)),
          pl.BlockSpec((N_CLASS, WINDOW * EMB), lambda i: (0, 0)),
          pl.BlockSpec((BM, N_CLASS), lambda i: (i, 0)),
      ],
      out_specs=pl.BlockSpec((BM, N_CLASS), lambda i: (i, 0)),
      out_shape=jax.ShapeDtypeStruct((BATCH, N_CLASS), jnp.float32),
  )(g, W, prev)


def _make_idx(xt, lo, n):
  return (xt[lo:lo + n]
          .reshape(n, NW, NB, CHUNK)
          .transpose(1, 0, 2, 3)
          .reshape(NW, n * NB, CHUNK))


@jax.jit
def kernel(x, table, W, b):
  xt = x.astype(jnp.int32).T
  g_a = _sc_gather_a(_make_idx(xt, 0, WSPLIT), table)
  part = _tc_matmul_a(g_a, W, b.reshape(1, N_CLASS))
  if WSPLIT == WINDOW:
    return part
  g_b = _sc_gather_b(_make_idx(xt, WSPLIT, WINDOW - WSPLIT), table)
  return _tc_matmul_b(g_b, W, part)
